# Initial kernel scaffold; baseline (speedup 1.0000x reference)
import jax
import jax.numpy as jnp
from jax.experimental import pallas as pl


def kernel(x, edge_index, W1, att_src1, att_dst1, b1, W2, att_src2, att_dst2, b2, W_skip, W3, att_src3, att_dst3, b3):
    raise NotImplementedError("write your pallas kernel here")


# speedup vs baseline: 13.8201x; 13.8201x over previous
import functools

import jax
import jax.numpy as jnp
from jax import lax
from jax.experimental import pallas as pl
from jax.experimental.pallas import tpu as pltpu
from jax.experimental.pallas import tpu_sc as plsc

N = 10000
Np = 10240
E = 320000
Etot = E + N
Ep = 331776
BLK = 128
NTILE = 16
EPT = Ep // NTILE
NBT = EPT // BLK
STRIPE = Np // NTILE
RB = 640
f32 = jnp.float32
i32 = jnp.int32


def _mesh():
    return plsc.VectorSubcoreMesh(core_axis_name="c", subcore_axis_name="s")


@functools.lru_cache(maxsize=None)
def _attn_kernel(H):
    HSC = H // 2
    scratch = [
        pltpu.VMEM((Np * H,), f32),
        pltpu.VMEM((Np * H,), f32),
        pltpu.VMEM((BLK,), i32),
        pltpu.VMEM((BLK,), i32),
        pltpu.VMEM((BLK,), f32),
        pltpu.VMEM((STRIPE,), f32),
    ] + [pltpu.VMEM_SHARED((Np,), f32) for _ in range(HSC)]

    def body(as_hbm, ad_hbm, src_hbm, dst_hbm, w_out, inv_out,
             as_t, ad_t, src_b, dst_b, w_b, den_t, *dens):
        core = lax.axis_index("c")
        sub = lax.axis_index("s")
        pltpu.sync_copy(as_hbm, as_t)
        pltpu.sync_copy(ad_hbm, ad_t)
        z16 = jnp.zeros((16,), f32)

        def zloop(i, _):
            den_t[pl.ds(i * 16, 16)] = z16
            return 0
        lax.fori_loop(0, STRIPE // 16, zloop, 0)
        for dh in dens:
            pltpu.sync_copy(den_t, dh.at[pl.ds(sub * STRIPE, STRIPE)])
        plsc.subcore_barrier()

        def block(b, _):
            base = sub * EPT + b * BLK
            pltpu.sync_copy(src_hbm.at[pl.ds(base, BLK)], src_b)
            pltpu.sync_copy(dst_hbm.at[pl.ds(base, BLK)], dst_b)
            for hl in range(HSC):
                h = core * HSC + hl

                def grp(g, _):
                    sv = src_b[pl.ds(g * 16, 16)]
                    dv = dst_b[pl.ds(g * 16, 16)]
                    a = plsc.load_gather(as_t, [sv * H + h])
                    d = plsc.load_gather(ad_t, [dv * H + h])
                    e = a + d
                    e = jnp.maximum(e, 0.2 * e)
                    w_b[pl.ds(g * 16, 16)] = jnp.exp(e)
                    return 0
                lax.fori_loop(0, BLK // 16, grp, 0)
                pltpu.sync_copy(w_b, w_out.at[pl.ds(h * Ep + base, BLK)])
                pltpu.sync_copy(w_b, dens[hl].at[dst_b], add=True)
            return 0
        lax.fori_loop(0, NBT, block, 0)
        plsc.subcore_barrier()

        for hl in range(HSC):
            h = core * HSC + hl
            pltpu.sync_copy(dens[hl].at[pl.ds(sub * STRIPE, STRIPE)], den_t)

            def invb(i, _):
                v = den_t[pl.ds(i * 16, 16)]
                den_t[pl.ds(i * 16, 16)] = 1.0 / (v + 1e-16)
                return 0
            lax.fori_loop(0, STRIPE // 16, invb, 0)
            pltpu.sync_copy(den_t, inv_out.at[pl.ds(h * Np + sub * STRIPE, STRIPE)])

    return pl.kernel(
        body,
        out_type=(jax.ShapeDtypeStruct((H * Ep,), f32),
                  jax.ShapeDtypeStruct((H * Np,), f32)),
        mesh=_mesh(),
        scratch_types=scratch,
        compiler_params=pltpu.CompilerParams(needs_layout_passes=False),
    )


@functools.lru_cache(maxsize=None)
def _agg_kernel(NC, CPH):
    CPS = NC // 2
    scratch = [
        pltpu.VMEM_SHARED((Np, BLK), f32),
        pltpu.VMEM((BLK,), i32),
        pltpu.VMEM((BLK,), i32),
        pltpu.VMEM((BLK,), f32),
        pltpu.VMEM((BLK, BLK), f32),
        pltpu.VMEM((64, BLK), f32),
        pltpu.SemaphoreType.DMA,
    ]

    def body(hp, src_hbm, dst_hbm, w_hbm, out_hbm,
             accum, src_b, dst_b, w_b, rows, zbuf, sem):
        core = lax.axis_index("c")
        sub = lax.axis_index("s")
        z16 = jnp.zeros((16,), f32)

        def zr(i, _):
            for j in range(BLK // 16):
                zbuf[i, pl.ds(j * 16, 16)] = z16
            return 0
        lax.fori_loop(0, 64, zr, 0)

        for ci in range(CPS):
            c = 2 * ci + core
            head = c // CPH
            for z in range(STRIPE // 64):
                pltpu.sync_copy(zbuf, accum.at[pl.ds(sub * STRIPE + z * 64, 64)])
            plsc.subcore_barrier()

            def block(b, _):
                base = sub * EPT + b * BLK
                pltpu.sync_copy(src_hbm.at[pl.ds(base, BLK)], src_b)
                pltpu.sync_copy(dst_hbm.at[pl.ds(base, BLK)], dst_b)
                pltpu.sync_copy(w_hbm.at[pl.ds(head * Ep + base, BLK)], w_b)
                off = c * Np

                def shift(g, _):
                    src_b[pl.ds(g * 16, 16)] = src_b[pl.ds(g * 16, 16)] + off
                    return 0
                lax.fori_loop(0, BLK // 16, shift, 0)
                pltpu.async_copy(hp.at[src_b], rows, sem).wait()

                def edge(r, _):
                    wv = plsc.load_gather(w_b, [jnp.full((16,), r, i32)])
                    for j in range(BLK // 16):
                        rows[r, pl.ds(j * 16, 16)] = rows[r, pl.ds(j * 16, 16)] * wv
                    return 0
                lax.fori_loop(0, BLK, edge, 0)
                pltpu.sync_copy(rows, accum.at[dst_b], add=True)
                return 0
            lax.fori_loop(0, NBT, block, 0)
            plsc.subcore_barrier()
            pltpu.sync_copy(accum.at[pl.ds(sub * STRIPE, STRIPE)],
                            out_hbm.at[pl.ds(c * Np + sub * STRIPE, STRIPE)])

    return pl.kernel(
        body,
        out_type=jax.ShapeDtypeStruct((NC * Np, BLK), f32),
        mesh=_mesh(),
        scratch_types=scratch,
        compiler_params=pltpu.CompilerParams(needs_layout_passes=False),
    )


def _full(shape):
    return pl.BlockSpec(shape, lambda i: tuple(0 for _ in shape))


def _tc1(x_p, W1, As1, Ad1):
    def body(x_ref, w_ref, asr, adr, hp_ref, a_ref, d_ref):
        h = jnp.dot(x_ref[...], w_ref[...], preferred_element_type=f32)
        a_ref[...] = jnp.dot(h, asr[...], preferred_element_type=f32)
        d_ref[...] = jnp.dot(h, adr[...], preferred_element_type=f32)
        for c in range(8):
            hp_ref[c] = h[:, c * BLK:(c + 1) * BLK]

    return pl.pallas_call(
        body,
        grid=(Np // RB,),
        in_specs=[pl.BlockSpec((RB, 128), lambda i: (i, 0)),
                  _full((128, 1024)), _full((1024, 4)), _full((1024, 4))],
        out_specs=[pl.BlockSpec((8, RB, BLK), lambda i: (0, i, 0)),
                   pl.BlockSpec((RB, 4), lambda i: (i, 0)),
                   pl.BlockSpec((RB, 4), lambda i: (i, 0))],
        out_shape=[jax.ShapeDtypeStruct((8, Np, BLK), f32),
                   jax.ShapeDtypeStruct((Np, 4), f32),
                   jax.ShapeDtypeStruct((Np, 4), f32)],
    )(x_p, W1, As1, Ad1)


def _tc_mid(agg, invt, b1m, W2, Wsk, As2, Ad2):
    def body(agg_ref, inv_ref, b_ref, w2_ref, wsk_ref, asr, adr,
             hp_ref, a_ref, d_ref, skip_ref, h1a):
        for c in range(8):
            hd = c // 2
            xv = agg_ref[c] * inv_ref[:, hd:hd + 1] + b_ref[c, :][None, :]
            h1a[:, c * BLK:(c + 1) * BLK] = jnp.where(xv > 0, xv, jnp.exp(xv) - 1.0)
        hv = h1a[...]
        h2 = jnp.dot(hv, w2_ref[...], preferred_element_type=f32)
        skip_ref[...] = jnp.dot(hv, wsk_ref[...], preferred_element_type=f32)
        a_ref[...] = jnp.dot(h2, asr[...], preferred_element_type=f32)
        d_ref[...] = jnp.dot(h2, adr[...], preferred_element_type=f32)
        for c in range(8):
            hp_ref[c] = h2[:, c * BLK:(c + 1) * BLK]

    return pl.pallas_call(
        body,
        grid=(Np // RB,),
        in_specs=[pl.BlockSpec((8, RB, BLK), lambda i: (0, i, 0)),
                  pl.BlockSpec((RB, 4), lambda i: (i, 0)),
                  _full((8, 128)), _full((1024, 1024)), _full((1024, 1024)),
                  _full((1024, 4)), _full((1024, 4))],
        out_specs=[pl.BlockSpec((8, RB, BLK), lambda i: (0, i, 0)),
                   pl.BlockSpec((RB, 4), lambda i: (i, 0)),
                   pl.BlockSpec((RB, 4), lambda i: (i, 0)),
                   pl.BlockSpec((RB, 1024), lambda i: (i, 0))],
        out_shape=[jax.ShapeDtypeStruct((8, Np, BLK), f32),
                   jax.ShapeDtypeStruct((Np, 4), f32),
                   jax.ShapeDtypeStruct((Np, 4), f32),
                   jax.ShapeDtypeStruct((Np, 1024), f32)],
        scratch_shapes=[pltpu.VMEM((RB, 1024), f32)],
    )(agg, invt, b1m, W2, Wsk, As2, Ad2)


def _tc3(agg, invt, b2m, skip, W3p, As3, Ad3):
    def body(agg_ref, inv_ref, b_ref, skip_ref, w3_ref, asr, adr,
             hp_ref, a_ref, d_ref, h2a):
        for c in range(8):
            hd = c // 2
            xv = agg_ref[c] * inv_ref[:, hd:hd + 1] + b_ref[c, :][None, :]
            h2a[:, c * BLK:(c + 1) * BLK] = (
                jnp.where(xv > 0, xv, jnp.exp(xv) - 1.0)
                + skip_ref[:, c * BLK:(c + 1) * BLK])
        hv = h2a[...]
        h3 = jnp.dot(hv, w3_ref[...], preferred_element_type=f32)
        a_ref[...] = jnp.dot(h3, asr[...], preferred_element_type=f32)
        d_ref[...] = jnp.dot(h3, adr[...], preferred_element_type=f32)
        for c in range(6):
            hp_ref[c] = h3[:, c * BLK:(c + 1) * BLK]

    return pl.pallas_call(
        body,
        grid=(Np // RB,),
        in_specs=[pl.BlockSpec((8, RB, BLK), lambda i: (0, i, 0)),
                  pl.BlockSpec((RB, 4), lambda i: (i, 0)),
                  _full((8, 128)),
                  pl.BlockSpec((RB, 1024), lambda i: (i, 0)),
                  _full((1024, 768)), _full((768, 6)), _full((768, 6))],
        out_specs=[pl.BlockSpec((6, RB, BLK), lambda i: (0, i, 0)),
                   pl.BlockSpec((RB, 6), lambda i: (i, 0)),
                   pl.BlockSpec((RB, 6), lambda i: (i, 0))],
        out_shape=[jax.ShapeDtypeStruct((6, Np, BLK), f32),
                   jax.ShapeDtypeStruct((Np, 6), f32),
                   jax.ShapeDtypeStruct((Np, 6), f32)],
        scratch_shapes=[pltpu.VMEM((RB, 1024), f32)],
    )(agg, invt, b2m, skip, W3p, As3, Ad3)


def _tc_final(agg, invt, b3m):
    def body(agg_ref, inv_ref, b_ref, out_ref):
        acc = agg_ref[0] * inv_ref[:, 0:1]
        for c in range(1, 6):
            acc = acc + agg_ref[c] * inv_ref[:, c:c + 1]
        out_ref[...] = acc * (1.0 / 6.0) + b_ref[...]

    return pl.pallas_call(
        body,
        grid=(Np // RB,),
        in_specs=[pl.BlockSpec((6, RB, BLK), lambda i: (0, i, 0)),
                  pl.BlockSpec((RB, 6), lambda i: (i, 0)),
                  _full((1, 128))],
        out_specs=pl.BlockSpec((RB, BLK), lambda i: (i, 0)),
        out_shape=jax.ShapeDtypeStruct((Np, BLK), f32),
    )(agg, invt, b3m)


def kernel(x, edge_index, W1, att_src1, att_dst1, b1, W2, att_src2, att_dst2,
           b2, W_skip, W3, att_src3, att_dst3, b3):
    x_p = jnp.zeros((Np, 128), f32).at[:N].set(x)
    loop = jnp.arange(N, dtype=i32)
    padi = (N + (jnp.arange(Ep - Etot, dtype=i32) % (Np - N))).astype(i32)
    src = jnp.concatenate([edge_index[0].astype(i32), loop, padi])
    dst = jnp.concatenate([edge_index[1].astype(i32), loop, padi])

    def bd(att):
        H = att.shape[0]
        return (att[:, :, None] * jnp.eye(H, dtype=f32)[:, None, :]).reshape(-1, H)

    As1, Ad1 = bd(att_src1), bd(att_dst1)
    As2, Ad2 = bd(att_src2), bd(att_dst2)
    As3 = bd(jnp.pad(att_src3, ((0, 0), (0, 7))))
    Ad3 = bd(jnp.pad(att_dst3, ((0, 0), (0, 7))))
    W3p = jnp.pad(W3.reshape(1024, 6, 121), ((0, 0), (0, 0), (0, 7))).reshape(1024, 768)
    b1m = b1.reshape(8, 128)
    b2m = b2.reshape(8, 128)
    b3m = jnp.pad(b3, (0, 7)).reshape(1, 128)

    hp1, as1, ad1 = _tc1(x_p, W1, As1, Ad1)
    w1, inv1 = _attn_kernel(4)(as1.reshape(-1), ad1.reshape(-1), src, dst)
    agg1 = _agg_kernel(8, 2)(hp1.reshape(8 * Np, BLK), src, dst, w1)
    hp2, as2, ad2, skip = _tc_mid(agg1.reshape(8, Np, BLK), inv1.reshape(4, Np).T,
                                  b1m, W2, W_skip, As2, Ad2)
    w2, inv2 = _attn_kernel(4)(as2.reshape(-1), ad2.reshape(-1), src, dst)
    agg2 = _agg_kernel(8, 2)(hp2.reshape(8 * Np, BLK), src, dst, w2)
    hp3, as3, ad3 = _tc3(agg2.reshape(8, Np, BLK), inv2.reshape(4, Np).T,
                         b2m, skip, W3p, As3, Ad3)
    w3, inv3 = _attn_kernel(6)(as3.reshape(-1), ad3.reshape(-1), src, dst)
    agg3 = _agg_kernel(6, 1)(hp3.reshape(6 * Np, BLK), src, dst, w3)
    outp = _tc_final(agg3.reshape(6, Np, BLK), inv3.reshape(6, Np).T, b3m)
    return outp[:N, :121]


# speedup vs baseline: 35.9727x; 2.6029x over previous
import functools

import jax
import jax.numpy as jnp
from jax import lax
from jax.experimental import pallas as pl
from jax.experimental.pallas import tpu as pltpu
from jax.experimental.pallas import tpu_sc as plsc

N = 10000
Np = 10240
E = 320000
Etot = E + N
Ep = 331776
BLK = 128
NTILE = 16
EPT = Ep // NTILE
NBT = EPT // BLK
STRIPE = Np // NTILE
RB = 640
EB = Ep // BLK
f32 = jnp.float32
i32 = jnp.int32


def _mesh():
    return plsc.VectorSubcoreMesh(core_axis_name="c", subcore_axis_name="s")


@functools.lru_cache(maxsize=None)
def _attn_kernel(H):
    HSC = H // 2
    SBA = 9
    SBE_A = SBA * BLK
    NSB = NBT // SBA
    scratch = (
        [pltpu.VMEM((Np * H,), f32),
         pltpu.VMEM((Np * H,), f32),
         pltpu.VMEM((SBE_A,), i32),
         pltpu.VMEM((SBE_A,), i32)]
        + [pltpu.VMEM((BLK,), i32)]
        + [pltpu.VMEM((SBE_A,), f32) for _ in range(HSC)]
        + [pltpu.VMEM_SHARED((Np,), f32) for _ in range(HSC)]
    )

    def body(as_hbm, ad_hbm, src_hbm, dst_hbm, w_out, inv_out,
             as_t, ad_t, src_sb, dst_sb, *rest):
        db = rest[0]
        wsbs = rest[1:1 + HSC]
        dens = rest[1 + HSC:1 + 2 * HSC]
        den_t = wsbs[0]
        core = lax.axis_index("c")
        sub = lax.axis_index("s")
        pltpu.sync_copy(as_hbm, as_t)
        pltpu.sync_copy(ad_hbm, ad_t)
        z16 = jnp.zeros((16,), f32)

        def zloop(i, _):
            den_t[pl.ds(i * 16, 16)] = z16
            return 0
        lax.fori_loop(0, STRIPE // 16, zloop, 0)
        for dh in dens:
            pltpu.sync_copy(den_t.at[pl.ds(0, STRIPE)], dh.at[pl.ds(sub * STRIPE, STRIPE)])
        plsc.subcore_barrier()

        def sblock(q, _):
            ebase = sub * EPT + q * SBE_A
            pltpu.sync_copy(src_hbm.at[pl.ds(ebase, SBE_A)], src_sb)
            pltpu.sync_copy(dst_hbm.at[pl.ds(ebase, SBE_A)], dst_sb)
            for hl in range(HSC):
                h = core * HSC + hl
                wsb = wsbs[hl]
                for kb in range(SBA):
                    for g in range(BLK // 16):
                        o = kb * BLK + g * 16
                        sv = src_sb[pl.ds(o, 16)]
                        dv = dst_sb[pl.ds(o, 16)]
                        a = plsc.load_gather(as_t, [sv * H + h])
                        d = plsc.load_gather(ad_t, [dv * H + h])
                        e = a + d
                        e = jnp.maximum(e, 0.2 * e)
                        wsb[pl.ds(o, 16)] = jnp.exp(e)
                    for gi in range(BLK // 16):
                        db[pl.ds(gi * 16, 16)] = dst_sb[pl.ds(kb * BLK + gi * 16, 16)]
                    pltpu.sync_copy(wsb.at[pl.ds(kb * BLK, BLK)],
                                    dens[hl].at[db], add=True)
                pltpu.sync_copy(wsb, w_out.at[pl.ds(h * Ep + ebase, SBE_A)])
            return 0
        lax.fori_loop(0, NSB, sblock, 0)
        plsc.subcore_barrier()

        for hl in range(HSC):
            h = core * HSC + hl
            pltpu.sync_copy(dens[hl].at[pl.ds(sub * STRIPE, STRIPE)],
                            den_t.at[pl.ds(0, STRIPE)])

            def invb(i, _):
                v = den_t[pl.ds(i * 16, 16)]
                den_t[pl.ds(i * 16, 16)] = 1.0 / (v + 1e-16)
                return 0
            lax.fori_loop(0, STRIPE // 16, invb, 0)
            pltpu.sync_copy(den_t.at[pl.ds(0, STRIPE)],
                            inv_out.at[pl.ds(h * Np + sub * STRIPE, STRIPE)])

    return pl.kernel(
        body,
        out_type=(jax.ShapeDtypeStruct((H * Ep,), f32),
                  jax.ShapeDtypeStruct((H * Np,), f32)),
        mesh=_mesh(),
        scratch_types=scratch,
        compiler_params=pltpu.CompilerParams(needs_layout_passes=False),
    )


@functools.lru_cache(maxsize=None)
def _agg_kernel(NC, CPH):
    CPS = NC // 2
    ABLK = 64
    NBA = EPT // ABLK
    SB = 54
    SBE = SB * ABLK
    NOB = NBA // SB
    scratch = [
        pltpu.VMEM_SHARED((Np, BLK), f32),
        pltpu.VMEM((SBE,), i32),
        pltpu.VMEM((SBE,), i32),
        pltpu.VMEM((SBE,), f32),
        pltpu.VMEM((ABLK,), i32),
        pltpu.VMEM((ABLK,), i32),
        pltpu.VMEM((ABLK, BLK), f32),
        pltpu.VMEM((ABLK, BLK), f32),
        pltpu.VMEM((ABLK, BLK), f32),
        pltpu.VMEM((ABLK, BLK), f32),
        pltpu.SemaphoreType.DMA,
        pltpu.SemaphoreType.DMA,
        pltpu.SemaphoreType.DMA,
        pltpu.SemaphoreType.DMA,
    ]

    def body(hp, src_hbm, dst_hbm, w_hbm, out_hbm, accum,
             src_sb, dst_sb, w_sb, db0, db1, g0, g1, s0, s1,
             gsem0, gsem1, ssem0, ssem1):
        core = lax.axis_index("c")
        sub = lax.axis_index("s")
        z16 = jnp.zeros((16,), f32)

        def issue_gather(k, gref, sem):
            pltpu.async_copy(hp.at[src_sb.at[pl.ds(k * ABLK, ABLK)]], gref, sem)

        def wait_gather(gref, sem):
            pltpu.make_async_copy(hp.at[src_sb.at[pl.ds(0, ABLK)]], gref, sem).wait()

        def issue_scatter(db, sref, sem):
            pltpu.async_copy(sref, accum.at[db], sem, add=True)

        def wait_scatter(sref, sem):
            pltpu.make_async_copy(sref, accum.at[db0], sem).wait()

        def scale(k, gref, sref):
            for g in range(ABLK // 16):
                wg = w_sb[pl.ds(k * ABLK + g * 16, 16)]
                for r2 in range(16):
                    wv = jnp.full((16,), wg[r2], f32)
                    r = g * 16 + r2
                    for j in range(BLK // 16):
                        sref[r, pl.ds(j * 16, 16)] = gref[r, pl.ds(j * 16, 16)] * wv

        def chunk_body(ci, _):
            c = 2 * ci + core
            head = c // CPH
            def zr(i, _):
                for j in range(BLK // 16):
                    s0[i, pl.ds(j * 16, 16)] = z16
                return 0
            lax.fori_loop(0, ABLK, zr, 0)
            for z in range(STRIPE // ABLK):
                pltpu.sync_copy(s0, accum.at[pl.ds(sub * STRIPE + z * ABLK, ABLK)])
            plsc.subcore_barrier()

            def outer(ob, _):
                ebase = sub * EPT + ob * SBE
                pltpu.sync_copy(src_hbm.at[pl.ds(ebase, SBE)], src_sb)
                pltpu.sync_copy(dst_hbm.at[pl.ds(ebase, SBE)], dst_sb)
                pltpu.sync_copy(w_hbm.at[pl.ds(head * Ep + ebase, SBE)], w_sb)
                off = c * Np

                def shift(i, _):
                    src_sb[pl.ds(i * 16, 16)] = src_sb[pl.ds(i * 16, 16)] + off
                    return 0
                lax.fori_loop(0, SBE // 16, shift, 0)

                issue_gather(0, g0, gsem0)
                issue_gather(1, g1, gsem1)

                def pair(p, _):
                    b0 = 2 * p
                    wait_gather(g0, gsem0)

                    @pl.when(p > 0)
                    def _():
                        wait_scatter(s0, ssem0)
                    scale(b0, g0, s0)
                    for gi in range(ABLK // 16):
                        db0[pl.ds(gi * 16, 16)] = dst_sb[pl.ds(b0 * ABLK + gi * 16, 16)]
                    issue_scatter(db0, s0, ssem0)

                    @pl.when(p < SB // 2 - 1)
                    def _():
                        issue_gather(b0 + 2, g0, gsem0)
                    wait_gather(g1, gsem1)

                    @pl.when(p > 0)
                    def _():
                        wait_scatter(s1, ssem1)
                    scale(b0 + 1, g1, s1)
                    for gi in range(ABLK // 16):
                        db1[pl.ds(gi * 16, 16)] = dst_sb[pl.ds((b0 + 1) * ABLK + gi * 16, 16)]
                    issue_scatter(db1, s1, ssem1)

                    @pl.when(p < SB // 2 - 1)
                    def _():
                        issue_gather(b0 + 3, g1, gsem1)
                    return 0
                lax.fori_loop(0, SB // 2, pair, 0)
                wait_scatter(s0, ssem0)
                wait_scatter(s1, ssem1)
                return 0
            lax.fori_loop(0, NOB, outer, 0)
            plsc.subcore_barrier()
            pltpu.sync_copy(accum.at[pl.ds(sub * STRIPE, STRIPE)],
                            out_hbm.at[pl.ds(c * Np + sub * STRIPE, STRIPE)])
            return 0
        lax.fori_loop(0, CPS, chunk_body, 0)

    return pl.kernel(
        body,
        out_type=jax.ShapeDtypeStruct((NC * Np, BLK), f32),
        mesh=_mesh(),
        scratch_types=scratch,
        compiler_params=pltpu.CompilerParams(needs_layout_passes=False),
    )


def _full(shape):
    return pl.BlockSpec(shape, lambda i: tuple(0 for _ in shape))


def _tc1(x_p, W1, As1, Ad1):
    def body(x_ref, w_ref, asr, adr, hp_ref, a_ref, d_ref):
        h = jnp.dot(x_ref[...], w_ref[...], preferred_element_type=f32)
        a_ref[...] = jnp.dot(h, asr[...], preferred_element_type=f32)
        d_ref[...] = jnp.dot(h, adr[...], preferred_element_type=f32)
        for c in range(8):
            hp_ref[c] = h[:, c * BLK:(c + 1) * BLK]

    return pl.pallas_call(
        body,
        grid=(Np // RB,),
        in_specs=[pl.BlockSpec((RB, 128), lambda i: (i, 0)),
                  _full((128, 1024)), _full((1024, 4)), _full((1024, 4))],
        out_specs=[pl.BlockSpec((8, RB, BLK), lambda i: (0, i, 0)),
                   pl.BlockSpec((RB, 4), lambda i: (i, 0)),
                   pl.BlockSpec((RB, 4), lambda i: (i, 0))],
        out_shape=[jax.ShapeDtypeStruct((8, Np, BLK), f32),
                   jax.ShapeDtypeStruct((Np, 4), f32),
                   jax.ShapeDtypeStruct((Np, 4), f32)],
    )(x_p, W1, As1, Ad1)


def _tc_mid(agg, invt, b1m, W2, Wsk, As2, Ad2):
    def body(agg_ref, inv_ref, b_ref, w2_ref, wsk_ref, asr, adr,
             hp_ref, a_ref, d_ref, skip_ref, h1a):
        for c in range(8):
            hd = c // 2
            xv = agg_ref[c] * inv_ref[:, hd:hd + 1] + b_ref[c, :][None, :]
            h1a[:, c * BLK:(c + 1) * BLK] = jnp.where(xv > 0, xv, jnp.exp(xv) - 1.0)
        hv = h1a[...]
        h2 = jnp.dot(hv, w2_ref[...], preferred_element_type=f32)
        skip_ref[...] = jnp.dot(hv, wsk_ref[...], preferred_element_type=f32)
        a_ref[...] = jnp.dot(h2, asr[...], preferred_element_type=f32)
        d_ref[...] = jnp.dot(h2, adr[...], preferred_element_type=f32)
        for c in range(8):
            hp_ref[c] = h2[:, c * BLK:(c + 1) * BLK]

    return pl.pallas_call(
        body,
        grid=(Np // RB,),
        in_specs=[pl.BlockSpec((8, RB, BLK), lambda i: (0, i, 0)),
                  pl.BlockSpec((RB, 4), lambda i: (i, 0)),
                  _full((8, 128)), _full((1024, 1024)), _full((1024, 1024)),
                  _full((1024, 4)), _full((1024, 4))],
        out_specs=[pl.BlockSpec((8, RB, BLK), lambda i: (0, i, 0)),
                   pl.BlockSpec((RB, 4), lambda i: (i, 0)),
                   pl.BlockSpec((RB, 4), lambda i: (i, 0)),
                   pl.BlockSpec((RB, 1024), lambda i: (i, 0))],
        out_shape=[jax.ShapeDtypeStruct((8, Np, BLK), f32),
                   jax.ShapeDtypeStruct((Np, 4), f32),
                   jax.ShapeDtypeStruct((Np, 4), f32),
                   jax.ShapeDtypeStruct((Np, 1024), f32)],
        scratch_shapes=[pltpu.VMEM((RB, 1024), f32)],
    )(agg, invt, b1m, W2, Wsk, As2, Ad2)


def _tc3(agg, invt, b2m, skip, W3p, As3, Ad3):
    def body(agg_ref, inv_ref, b_ref, skip_ref, w3_ref, asr, adr,
             hp_ref, a_ref, d_ref, h2a):
        for c in range(8):
            hd = c // 2
            xv = agg_ref[c] * inv_ref[:, hd:hd + 1] + b_ref[c, :][None, :]
            h2a[:, c * BLK:(c + 1) * BLK] = (
                jnp.where(xv > 0, xv, jnp.exp(xv) - 1.0)
                + skip_ref[:, c * BLK:(c + 1) * BLK])
        hv = h2a[...]
        h3 = jnp.dot(hv, w3_ref[...], preferred_element_type=f32)
        a_ref[...] = jnp.dot(h3, asr[...], preferred_element_type=f32)
        d_ref[...] = jnp.dot(h3, adr[...], preferred_element_type=f32)
        for c in range(6):
            hp_ref[c] = h3[:, c * BLK:(c + 1) * BLK]

    return pl.pallas_call(
        body,
        grid=(Np // RB,),
        in_specs=[pl.BlockSpec((8, RB, BLK), lambda i: (0, i, 0)),
                  pl.BlockSpec((RB, 4), lambda i: (i, 0)),
                  _full((8, 128)),
                  pl.BlockSpec((RB, 1024), lambda i: (i, 0)),
                  _full((1024, 768)), _full((768, 6)), _full((768, 6))],
        out_specs=[pl.BlockSpec((6, RB, BLK), lambda i: (0, i, 0)),
                   pl.BlockSpec((RB, 6), lambda i: (i, 0)),
                   pl.BlockSpec((RB, 6), lambda i: (i, 0))],
        out_shape=[jax.ShapeDtypeStruct((6, Np, BLK), f32),
                   jax.ShapeDtypeStruct((Np, 6), f32),
                   jax.ShapeDtypeStruct((Np, 6), f32)],
        scratch_shapes=[pltpu.VMEM((RB, 1024), f32)],
    )(agg, invt, b2m, skip, W3p, As3, Ad3)


def _tc_final(agg, invt, b3m):
    def body(agg_ref, inv_ref, b_ref, out_ref):
        acc = agg_ref[0] * inv_ref[:, 0:1]
        for c in range(1, 6):
            acc = acc + agg_ref[c] * inv_ref[:, c:c + 1]
        out_ref[...] = acc * (1.0 / 6.0) + b_ref[...]

    return pl.pallas_call(
        body,
        grid=(Np // RB,),
        in_specs=[pl.BlockSpec((6, RB, BLK), lambda i: (0, i, 0)),
                  pl.BlockSpec((RB, 6), lambda i: (i, 0)),
                  _full((1, 128))],
        out_specs=pl.BlockSpec((RB, BLK), lambda i: (i, 0)),
        out_shape=jax.ShapeDtypeStruct((Np, BLK), f32),
    )(agg, invt, b3m)


def kernel(x, edge_index, W1, att_src1, att_dst1, b1, W2, att_src2, att_dst2,
           b2, W_skip, W3, att_src3, att_dst3, b3):
    x_p = jnp.zeros((Np, 128), f32).at[:N].set(x)
    loop = jnp.arange(N, dtype=i32)
    padi = (N + (jnp.arange(Ep - Etot, dtype=i32) % (Np - N))).astype(i32)
    src = jnp.concatenate([edge_index[0].astype(i32), loop, padi])
    dst = jnp.concatenate([edge_index[1].astype(i32), loop, padi])

    def bd(att):
        H = att.shape[0]
        return (att[:, :, None] * jnp.eye(H, dtype=f32)[:, None, :]).reshape(-1, H)

    As1, Ad1 = bd(att_src1), bd(att_dst1)
    As2, Ad2 = bd(att_src2), bd(att_dst2)
    As3 = bd(jnp.pad(att_src3, ((0, 0), (0, 7))))
    Ad3 = bd(jnp.pad(att_dst3, ((0, 0), (0, 7))))
    W3p = jnp.pad(W3.reshape(1024, 6, 121), ((0, 0), (0, 0), (0, 7))).reshape(1024, 768)
    b1m = b1.reshape(8, 128)
    b2m = b2.reshape(8, 128)
    b3m = jnp.pad(b3, (0, 7)).reshape(1, 128)

    hp1, as1, ad1 = _tc1(x_p, W1, As1, Ad1)
    w1, inv1 = _attn_kernel(4)(as1.reshape(-1), ad1.reshape(-1), src, dst)
    agg1 = _agg_kernel(8, 2)(hp1.reshape(8 * Np, BLK), src, dst, w1)
    hp2, as2, ad2, skip = _tc_mid(agg1.reshape(8, Np, BLK), inv1.reshape(4, Np).T,
                                  b1m, W2, W_skip, As2, Ad2)
    w2, inv2 = _attn_kernel(4)(as2.reshape(-1), ad2.reshape(-1), src, dst)
    agg2 = _agg_kernel(8, 2)(hp2.reshape(8 * Np, BLK), src, dst, w2)
    hp3, as3, ad3 = _tc3(agg2.reshape(8, Np, BLK), inv2.reshape(4, Np).T,
                         b2m, skip, W3p, As3, Ad3)
    w3, inv3 = _attn_kernel(6)(as3.reshape(-1), ad3.reshape(-1), src, dst)
    agg3 = _agg_kernel(6, 1)(hp3.reshape(6 * Np, BLK), src, dst, w3)
    outp = _tc_final(agg3.reshape(6, Np, BLK), inv3.reshape(6, Np).T, b3m)
    return outp[:N, :121]
